# Initial kernel scaffold; baseline (speedup 1.0000x reference)
import jax
import jax.numpy as jnp
from jax.experimental import pallas as pl


def kernel(x, fc1_w, fc1_b, fc2_w, fc2_b, k1_w, k1_b, k2_w, k2_b, w1_w, w1_b, w2_w, w2_b):
    raise NotImplementedError("write your pallas kernel here")


# speedup vs baseline: 10.7349x; 10.7349x over previous
import functools

import jax
import jax.numpy as jnp
from jax.experimental import pallas as pl

_K_MIN = 1.0
_K_MAX = 12.0
_ALPHA = 12.0
_TOPK = 12
_NEG = -1e30


def _body(x_ref, fc1_w_ref, fc1_b_ref, fc2_w_ref, fc2_b_ref,
          k1_w_ref, k1_b_ref, k2_w_ref, k2_b_ref,
          w1_w_ref, w1_b_ref, w2_w_ref, w2_b_ref, o_ref):
    xb = x_ref[0]
    n = xb.shape[0]

    pooled = jnp.mean(xb, axis=0, keepdims=True)
    t = jnp.maximum(
        jnp.dot(pooled, k1_w_ref[...], preferred_element_type=jnp.float32)
        + k1_b_ref[...], 0.0)
    kl = jnp.dot(t, k2_w_ref[...], preferred_element_type=jnp.float32) + k2_b_ref[...]
    kc = _K_MIN + jax.nn.sigmoid(kl) * (_K_MAX - _K_MIN)
    t2 = jnp.maximum(
        jnp.dot(pooled, w1_w_ref[...], preferred_element_type=jnp.float32)
        + w1_b_ref[...], 0.0)
    wl = jnp.dot(t2, w2_w_ref[...], preferred_element_type=jnp.float32) + w2_b_ref[...]

    k_i = [kc[0, i] for i in range(3)]
    l_i = [wl[0, i] for i in range(3)]
    lmax = jnp.maximum(jnp.maximum(l_i[0], l_i[1]), l_i[2])
    e_i = [jnp.exp(l - lmax) for l in l_i]
    esum = e_i[0] + e_i[1] + e_i[2]
    w_i = [e / esum for e in e_i]

    h1 = jnp.maximum(
        jnp.dot(xb, fc1_w_ref[...], preferred_element_type=jnp.float32)
        + fc1_b_ref[...], 0.0)
    h = jnp.dot(h1, fc2_w_ref[...], preferred_element_type=jnp.float32) + fc2_b_ref[...]

    gram = jnp.dot(h, h.T, preferred_element_type=jnp.float32)
    rows = jax.lax.broadcasted_iota(jnp.int32, (n, n), 0)
    cols = jax.lax.broadcasted_iota(jnp.int32, (n, n), 1)
    eye = (rows == cols).astype(jnp.float32)
    sq_row = jnp.sum(gram * eye, axis=0, keepdims=True)
    logits = 2.0 * gram - sq_row

    work = logits
    rank = jnp.zeros((n, n), jnp.float32)
    vals = []
    for j in range(_TOPK):
        cur = jnp.max(work, axis=1, keepdims=True)
        is_max = work >= cur
        first = jnp.min(jnp.where(is_max, cols, n), axis=1, keepdims=True)
        sel = cols == first
        rank = jnp.where(sel, float(j + 1), rank)
        vals.append(cur)
        work = jnp.where(sel, _NEG, work)

    v0 = vals[0]
    s = [jnp.exp(v - v0) for v in vals]
    ssum = functools.reduce(jnp.add, s)
    p = [sj / ssum for sj in s]
    m = [[jax.nn.sigmoid(_ALPHA * (k_i[i] - float(j + 1))) for j in range(_TOPK)]
         for i in range(3)]
    den = [functools.reduce(jnp.add, [p[j] * m[i][j] for j in range(_TOPK)]) + 1e-8
           for i in range(3)]
    coef = [w_i[0] * m[0][j] / den[0] + w_i[1] * m[1][j] / den[1]
            + w_i[2] * m[2][j] / den[2] for j in range(_TOPK)]
    attn = jnp.zeros((n, n), jnp.float32)
    for j in range(_TOPK):
        attn = attn + jnp.where(rank == float(j + 1), p[j] * coef[j], 0.0)

    o_ref[0] = jnp.dot(attn, h, preferred_element_type=jnp.float32)


def kernel(x, fc1_w, fc1_b, fc2_w, fc2_b, k1_w, k1_b, k2_w, k2_b,
           w1_w, w1_b, w2_w, w2_b):
    B, N, C = x.shape
    H = fc1_w.shape[1]
    k2_wp = jnp.pad(k2_w, ((0, 0), (0, 128 - k2_w.shape[1])))
    k2_bp = jnp.pad(k2_b, (0, 128 - k2_b.shape[0])).reshape(1, 128)
    w2_wp = jnp.pad(w2_w, ((0, 0), (0, 128 - w2_w.shape[1])))
    w2_bp = jnp.pad(w2_b, (0, 128 - w2_b.shape[0])).reshape(1, 128)

    const = lambda shape: pl.BlockSpec(shape, lambda b: (0,) * len(shape))
    return pl.pallas_call(
        _body,
        grid=(B,),
        in_specs=[
            pl.BlockSpec((1, N, C), lambda b: (b, 0, 0)),
            const((C, H)), const((1, H)),
            const((H, C)), const((1, C)),
            const((C, 128)), const((1, 128)),
            const((128, 128)), const((1, 128)),
            const((C, 128)), const((1, 128)),
            const((128, 128)), const((1, 128)),
        ],
        out_specs=pl.BlockSpec((1, N, C), lambda b: (b, 0, 0)),
        out_shape=jax.ShapeDtypeStruct((B, N, C), jnp.float32),
    )(x, fc1_w, fc1_b.reshape(1, H), fc2_w, fc2_b.reshape(1, C),
      k1_w, k1_b.reshape(1, 128), k2_wp, k2_bp,
      w1_w, w1_b.reshape(1, 128), w2_wp, w2_bp)


# speedup vs baseline: 14.3706x; 1.3387x over previous
import functools

import jax
import jax.numpy as jnp
from jax.experimental import pallas as pl

_K_MIN = 1.0
_K_MAX = 12.0
_ALPHA = 12.0
_TOPK = 12
_NEG = -1e30


def _body(x_ref, fc1_w_ref, fc1_b_ref, fc2_w_ref, fc2_b_ref,
          k1_w_ref, k1_b_ref, k2_w_ref, k2_b_ref,
          w1_w_ref, w1_b_ref, w2_w_ref, w2_b_ref, o_ref):
    xb = x_ref[0]
    n = xb.shape[0]

    pooled = jnp.mean(xb, axis=0, keepdims=True)
    t = jnp.maximum(
        jnp.dot(pooled, k1_w_ref[...], preferred_element_type=jnp.float32)
        + k1_b_ref[...], 0.0)
    kl = jnp.dot(t, k2_w_ref[...], preferred_element_type=jnp.float32) + k2_b_ref[...]
    kc = _K_MIN + jax.nn.sigmoid(kl) * (_K_MAX - _K_MIN)
    t2 = jnp.maximum(
        jnp.dot(pooled, w1_w_ref[...], preferred_element_type=jnp.float32)
        + w1_b_ref[...], 0.0)
    wl = jnp.dot(t2, w2_w_ref[...], preferred_element_type=jnp.float32) + w2_b_ref[...]

    k_i = [kc[0, i] for i in range(3)]
    l_i = [wl[0, i] for i in range(3)]
    lmax = jnp.maximum(jnp.maximum(l_i[0], l_i[1]), l_i[2])
    e_i = [jnp.exp(l - lmax) for l in l_i]
    esum = e_i[0] + e_i[1] + e_i[2]
    w_i = [e / esum for e in e_i]

    h1 = jnp.maximum(
        jnp.dot(xb, fc1_w_ref[...], preferred_element_type=jnp.float32)
        + fc1_b_ref[...], 0.0)
    h = jnp.dot(h1, fc2_w_ref[...], preferred_element_type=jnp.float32) + fc2_b_ref[...]

    gram = jnp.dot(h, h.T, preferred_element_type=jnp.float32)
    rows = jax.lax.broadcasted_iota(jnp.int32, (n, n), 0)
    cols = jax.lax.broadcasted_iota(jnp.int32, (n, n), 1)
    eye = (rows == cols).astype(jnp.float32)
    sq_row = jnp.sum(gram * eye, axis=0, keepdims=True)
    logits = 2.0 * gram - sq_row

    m = [[jax.nn.sigmoid(_ALPHA * (k_i[i] - float(j + 1))) for j in range(_TOPK)]
         for i in range(3)]
    work = logits
    acc = [jnp.zeros((n, n), jnp.float32) for _ in range(3)]
    v0 = None
    ssum = None
    for j in range(_TOPK):
        cur = jnp.max(work, axis=1, keepdims=True)
        sel = work >= cur
        if j == 0:
            v0 = cur
            s = jnp.ones((n, 1), jnp.float32)
            ssum = s
        else:
            s = jnp.exp(cur - v0)
            ssum = ssum + s
        for i in range(3):
            acc[i] = acc[i] + jnp.where(sel, s * m[i][j], 0.0)
        work = jnp.where(sel, _NEG, work)

    attn = functools.reduce(jnp.add, [
        (w_i[i] / (jnp.sum(acc[i], axis=1, keepdims=True) + 1e-8 * ssum)) * acc[i]
        for i in range(3)])

    o_ref[0] = jnp.dot(attn, h, preferred_element_type=jnp.float32)


def kernel(x, fc1_w, fc1_b, fc2_w, fc2_b, k1_w, k1_b, k2_w, k2_b,
           w1_w, w1_b, w2_w, w2_b):
    B, N, C = x.shape
    H = fc1_w.shape[1]
    k2_wp = jnp.pad(k2_w, ((0, 0), (0, 128 - k2_w.shape[1])))
    k2_bp = jnp.pad(k2_b, (0, 128 - k2_b.shape[0])).reshape(1, 128)
    w2_wp = jnp.pad(w2_w, ((0, 0), (0, 128 - w2_w.shape[1])))
    w2_bp = jnp.pad(w2_b, (0, 128 - w2_b.shape[0])).reshape(1, 128)

    const = lambda shape: pl.BlockSpec(shape, lambda b: (0,) * len(shape))
    return pl.pallas_call(
        _body,
        grid=(B,),
        in_specs=[
            pl.BlockSpec((1, N, C), lambda b: (b, 0, 0)),
            const((C, H)), const((1, H)),
            const((H, C)), const((1, C)),
            const((C, 128)), const((1, 128)),
            const((128, 128)), const((1, 128)),
            const((C, 128)), const((1, 128)),
            const((128, 128)), const((1, 128)),
        ],
        out_specs=pl.BlockSpec((1, N, C), lambda b: (b, 0, 0)),
        out_shape=jax.ShapeDtypeStruct((B, N, C), jnp.float32),
    )(x, fc1_w, fc1_b.reshape(1, H), fc2_w, fc2_b.reshape(1, C),
      k1_w, k1_b.reshape(1, 128), k2_wp, k2_bp,
      w1_w, w1_b.reshape(1, 128), w2_wp, w2_bp)
